# baseline (speedup 1.0000x reference)
import functools

import jax
import jax.numpy as jnp
import numpy as np
from jax import lax
from jax.experimental import pallas as pl
from jax.experimental.pallas import tpu as pltpu
from jax.experimental.pallas import tpu_sc as plsc

BATCH = 16384
HIST = 200
EMBED_DIM = 6
NC, NS = 2, 16
NW = NC * NS
L = 16

CW = 256
CPR = BATCH // CW
NCHUNKS = HIST * CPR
CPW = NCHUNKS // NW
NBUF = 8

_PATS = np.tile(np.arange(EMBED_DIM, dtype=np.int32)[:, None], (1, L))


def _permute(vec, idx):
    return lax.gather(
        vec, idx[:, None],
        dimension_numbers=lax.GatherDimensionNumbers(
            offset_dims=(), collapsed_slice_dims=(0,), start_index_map=(0,)),
        slice_sizes=(1,),
        mode=lax.GatherScatterMode.PROMISE_IN_BOUNDS)


def _sc_lookup(idx_t, tab_pad, pats):
    mesh = plsc.VectorSubcoreMesh(core_axis_name="c", subcore_axis_name="s")

    @functools.partial(
        pl.kernel,
        mesh=mesh,
        out_type=jax.ShapeDtypeStruct((EMBED_DIM, HIST, BATCH), jnp.float32),
        scratch_types=(
            [pltpu.VMEM((NBUF, CW), jnp.int32),
             pltpu.VMEM((NBUF, EMBED_DIM, CW), jnp.float32),
             pltpu.VMEM((2 * L,), jnp.float32),
             pltpu.VMEM(_PATS.shape, jnp.int32)]
            + [pltpu.SemaphoreType.DMA] * (2 * NBUF)
        ),
    )
    def k(idx_hbm, tab_hbm, pats_hbm, out_hbm, idx_v, out_v, tab_v, pats_v,
          *sems):
        isem = sems[:NBUF]
        osem = sems[NBUF:]
        wid = lax.axis_index("s") * NC + lax.axis_index("c")
        k0 = wid * CPW

        pltpu.sync_copy(tab_hbm, tab_v)
        pltpu.sync_copy(pats_hbm, pats_v)

        t0 = tab_v[pl.ds(0, L)]
        t1 = tab_v[pl.ds(L, L)]
        w0 = [_permute(t0, pats_v[c, :]) for c in range(EMBED_DIM)]
        w1 = [_permute(t1, pats_v[c, :]) for c in range(EMBED_DIM)]

        def in_copy(kk, b):
            t = kk // CPR
            col = (kk % CPR) * CW
            return pltpu.make_async_copy(
                idx_hbm.at[t, pl.ds(col, CW)], idx_v.at[b], isem[b])

        def out_copy(kk, b, c):
            t = kk // CPR
            col = (kk % CPR) * CW
            return pltpu.make_async_copy(
                out_v.at[b, c], out_hbm.at[c, t, pl.ds(col, CW)], osem[b])

        def compute(kk, b):
            def vec_body(j, c2):
                iv = idx_v[b, pl.ds(j * L, L)]
                m = iv == 1
                for c in range(EMBED_DIM):
                    out_v[b, c, pl.ds(j * L, L)] = jnp.where(m, w1[c], w0[c])
                return c2

            lax.fori_loop(0, CW // L, vec_body, 0)

        for b in range(NBUF):
            in_copy(k0 + b, b).start()

        def step_body(s, carry):
            for b in range(NBUF):
                kk = k0 + s * NBUF + b
                in_copy(kk, b).wait()

                @pl.when(s > 0)
                def _wait_out():
                    for c in range(EMBED_DIM):
                        out_copy(kk, b, c).wait()

                compute(kk, b)

                for c in range(EMBED_DIM):
                    out_copy(kk, b, c).start()

                @pl.when(s * NBUF + b + NBUF < CPW)
                def _prefetch():
                    in_copy(kk + NBUF, b).start()
            return carry

        lax.fori_loop(0, CPW // NBUF, step_body, 0)

        for b in range(NBUF):
            kk = k0 + CPW - NBUF + b
            for c in range(EMBED_DIM):
                out_copy(kk, b, c).wait()

    return k(idx_t, tab_pad, pats)


def kernel(indices, table):
    idx_t = indices.astype(jnp.int32).T
    tab_pad = jnp.pad(table, ((0, 0), (0, L - EMBED_DIM))).reshape(-1)
    pats = jnp.asarray(_PATS)
    planes = _sc_lookup(idx_t, tab_pad, pats)
    return planes.transpose(2, 1, 0)


# speedup vs baseline: 1.1807x; 1.1807x over previous
import functools

import jax
import jax.numpy as jnp
import numpy as np
from jax import lax
from jax.experimental import pallas as pl
from jax.experimental.pallas import tpu as pltpu
from jax.experimental.pallas import tpu_sc as plsc

BATCH = 16384
HIST = 200
EMBED_DIM = 6
NC, NS = 2, 16
NW = NC * NS
L = 16

CW = 512
CPR = BATCH // CW
NCHUNKS = HIST * CPR
CPW = NCHUNKS // NW
NBUF = 8

_PATS = np.tile(np.arange(EMBED_DIM, dtype=np.int32)[:, None], (1, L))


def _permute(vec, idx):
    return lax.gather(
        vec, idx[:, None],
        dimension_numbers=lax.GatherDimensionNumbers(
            offset_dims=(), collapsed_slice_dims=(0,), start_index_map=(0,)),
        slice_sizes=(1,),
        mode=lax.GatherScatterMode.PROMISE_IN_BOUNDS)


def _sc_lookup(idx_t, tab_pad, pats):
    mesh = plsc.VectorSubcoreMesh(core_axis_name="c", subcore_axis_name="s")

    @functools.partial(
        pl.kernel,
        mesh=mesh,
        out_type=jax.ShapeDtypeStruct((EMBED_DIM, HIST, BATCH), jnp.float32),
        scratch_types=(
            [pltpu.VMEM((NBUF, CW), jnp.int32),
             pltpu.VMEM((NBUF, EMBED_DIM, CW), jnp.float32),
             pltpu.VMEM((2 * L,), jnp.float32),
             pltpu.VMEM(_PATS.shape, jnp.int32)]
            + [pltpu.SemaphoreType.DMA] * (2 * NBUF)
        ),
    )
    def k(idx_hbm, tab_hbm, pats_hbm, out_hbm, idx_v, out_v, tab_v, pats_v,
          *sems):
        isem = sems[:NBUF]
        osem = sems[NBUF:]
        wid = lax.axis_index("s") * NC + lax.axis_index("c")
        k0 = wid * CPW

        pltpu.sync_copy(tab_hbm, tab_v)
        pltpu.sync_copy(pats_hbm, pats_v)

        t0 = tab_v[pl.ds(0, L)]
        t1 = tab_v[pl.ds(L, L)]
        w0 = [_permute(t0, pats_v[c, :]) for c in range(EMBED_DIM)]
        w1 = [_permute(t1, pats_v[c, :]) for c in range(EMBED_DIM)]

        def in_copy(kk, b):
            t = kk // CPR
            col = (kk % CPR) * CW
            return pltpu.make_async_copy(
                idx_hbm.at[t, pl.ds(col, CW)], idx_v.at[b], isem[b])

        def out_copy(kk, b, c):
            t = kk // CPR
            col = (kk % CPR) * CW
            return pltpu.make_async_copy(
                out_v.at[b, c], out_hbm.at[c, t, pl.ds(col, CW)], osem[b])

        def compute(kk, b):
            def vec_body(j, c2):
                iv = idx_v[b, pl.ds(j * L, L)]
                m = iv == 1
                for c in range(EMBED_DIM):
                    out_v[b, c, pl.ds(j * L, L)] = jnp.where(m, w1[c], w0[c])
                return c2

            lax.fori_loop(0, CW // L, vec_body, 0)

        for b in range(NBUF):
            in_copy(k0 + b, b).start()

        def step_body(s, carry):
            for b in range(NBUF):
                kk = k0 + s * NBUF + b
                in_copy(kk, b).wait()

                @pl.when(s > 0)
                def _wait_out():
                    for c in range(EMBED_DIM):
                        out_copy(kk, b, c).wait()

                compute(kk, b)

                for c in range(EMBED_DIM):
                    out_copy(kk, b, c).start()

                @pl.when(s * NBUF + b + NBUF < CPW)
                def _prefetch():
                    in_copy(kk + NBUF, b).start()
            return carry

        lax.fori_loop(0, CPW // NBUF, step_body, 0)

        for b in range(NBUF):
            kk = k0 + CPW - NBUF + b
            for c in range(EMBED_DIM):
                out_copy(kk, b, c).wait()

    return k(idx_t, tab_pad, pats)


def kernel(indices, table):
    idx_t = indices.astype(jnp.int32).T
    tab_pad = jnp.pad(table, ((0, 0), (0, L - EMBED_DIM))).reshape(-1)
    pats = jnp.asarray(_PATS)
    planes = _sc_lookup(idx_t, tab_pad, pats)
    return planes.transpose(2, 1, 0)


# speedup vs baseline: 1.1839x; 1.0027x over previous
import functools

import jax
import jax.numpy as jnp
import numpy as np
from jax import lax
from jax.experimental import pallas as pl
from jax.experimental.pallas import tpu as pltpu
from jax.experimental.pallas import tpu_sc as plsc

BATCH = 16384
HIST = 200
EMBED_DIM = 6
NC, NS = 2, 16
NW = NC * NS
L = 16

CW = 512
CPR = BATCH // CW
NCHUNKS = HIST * CPR
CPW = NCHUNKS // NW
NBUF = 8

_PATS = np.tile(np.arange(EMBED_DIM, dtype=np.int32)[:, None], (1, L))


def _permute(vec, idx):
    return lax.gather(
        vec, idx[:, None],
        dimension_numbers=lax.GatherDimensionNumbers(
            offset_dims=(), collapsed_slice_dims=(0,), start_index_map=(0,)),
        slice_sizes=(1,),
        mode=lax.GatherScatterMode.PROMISE_IN_BOUNDS)


def _sc_lookup(idx_t, tab_pad, pats):
    mesh = plsc.VectorSubcoreMesh(core_axis_name="c", subcore_axis_name="s")

    @functools.partial(
        pl.kernel,
        mesh=mesh,
        out_type=jax.ShapeDtypeStruct((EMBED_DIM, HIST, BATCH), jnp.float32),
        scratch_types=(
            [pltpu.VMEM((NBUF, CW), jnp.int32),
             pltpu.VMEM((NBUF, EMBED_DIM, CW), jnp.float32),
             pltpu.VMEM((2 * L,), jnp.float32),
             pltpu.VMEM(_PATS.shape, jnp.int32)]
            + [pltpu.SemaphoreType.DMA] * (2 * NBUF)
        ),
    )
    def k(idx_hbm, tab_hbm, pats_hbm, out_hbm, idx_v, out_v, tab_v, pats_v,
          *sems):
        isem = sems[:NBUF]
        osem = sems[NBUF:]
        wid = lax.axis_index("c") * NS + lax.axis_index("s")
        k0 = wid * CPW

        pltpu.sync_copy(tab_hbm, tab_v)
        pltpu.sync_copy(pats_hbm, pats_v)

        t0 = tab_v[pl.ds(0, L)]
        t1 = tab_v[pl.ds(L, L)]
        w0 = [_permute(t0, pats_v[c, :]) for c in range(EMBED_DIM)]
        w1 = [_permute(t1, pats_v[c, :]) for c in range(EMBED_DIM)]

        def in_copy(kk, b):
            t = kk // CPR
            col = (kk % CPR) * CW
            return pltpu.make_async_copy(
                idx_hbm.at[t, pl.ds(col, CW)], idx_v.at[b], isem[b])

        def out_copy(kk, b, c):
            t = kk // CPR
            col = (kk % CPR) * CW
            return pltpu.make_async_copy(
                out_v.at[b, c], out_hbm.at[c, t, pl.ds(col, CW)], osem[b])

        def compute(kk, b):
            def vec_body(j, c2):
                iv = idx_v[b, pl.ds(j * L, L)]
                m = iv == 1
                for c in range(EMBED_DIM):
                    out_v[b, c, pl.ds(j * L, L)] = jnp.where(m, w1[c], w0[c])
                return c2

            lax.fori_loop(0, CW // L, vec_body, 0)

        for b in range(NBUF):
            in_copy(k0 + b, b).start()

        def step_body(s, carry):
            for b in range(NBUF):
                kk = k0 + s * NBUF + b
                in_copy(kk, b).wait()

                @pl.when(s > 0)
                def _wait_out():
                    for c in range(EMBED_DIM):
                        out_copy(kk, b, c).wait()

                compute(kk, b)

                for c in range(EMBED_DIM):
                    out_copy(kk, b, c).start()

                @pl.when(s * NBUF + b + NBUF < CPW)
                def _prefetch():
                    in_copy(kk + NBUF, b).start()
            return carry

        lax.fori_loop(0, CPW // NBUF, step_body, 0)

        for b in range(NBUF):
            kk = k0 + CPW - NBUF + b
            for c in range(EMBED_DIM):
                out_copy(kk, b, c).wait()

    return k(idx_t, tab_pad, pats)


def kernel(indices, table):
    idx_t = indices.astype(jnp.int32).T
    tab_pad = jnp.pad(table, ((0, 0), (0, L - EMBED_DIM))).reshape(-1)
    pats = jnp.asarray(_PATS)
    planes = _sc_lookup(idx_t, tab_pad, pats)
    return planes.transpose(2, 1, 0)
